# baseline (speedup 1.0000x reference)
import functools

import jax
import jax.numpy as jnp
from jax import lax
from jax.experimental import pallas as pl
from jax.experimental.pallas import tpu as pltpu
from jax.experimental.pallas import tpu_sc as plsc

_B = 16384
_D = 64
_L = 16
_NROW = 1000000
_NBLK = (_NROW + 127) // 128
_SPAD = _B + 8

_info = plsc.get_sparse_core_info()
_NC, _NS = _info.num_cores, _info.num_subcores
_NW = _NC * _NS
_BLKW = 245
_NPAIR = 123
_CAPC = 384
_CHK = _CAPC // 128


def _sweep(idx_hbm, tab_t, staged, idx_all, win0, win1, ul, bl, todo, cols,
           blist3, sem0, sem1, sem_s, start):
    iota = lax.iota(jnp.int32, _L)

    pltpu.sync_copy(idx_hbm, idx_all)

    def scan_step(j, cnt):
        uvec = idx_all[pl.ds(j * _L, _L)]
        blk = lax.shift_right_logical(uvec, 7)
        m = (blk >= start) & (blk < start + 2 * _NPAIR)
        plsc.store_compressed(ul.at[pl.ds(cnt, _L)], uvec, mask=m)
        plsc.store_compressed(bl.at[pl.ds(cnt, _L)], iota + (j * _L), mask=m)
        return cnt + plsc.all_reduce_population_count(m)[0]

    n_match = lax.fori_loop(0, _B // _L, scan_step, 0, unroll=False)
    n_scan = lax.div(n_match + (_L - 1), _L)

    def fetch(t, buf, sem):
        tc = jnp.minimum(t, _NBLK - 1)
        off = pl.multiple_of(tc * 128, 128)
        return pltpu.async_copy(tab_t.at[:, pl.ds(off, 128)], buf, sem)

    def flush(cnt2):
        def pad(i, c):
            k = i // (128 // _L)
            col = (i % (128 // _L)) * _L
            jvec = iota + (k * 128 + col)
            old = blist3[k, pl.ds(col, _L)]
            blist3[k, pl.ds(col, _L)] = jnp.where(jvec < cnt2, old, _B)
            return c

        lax.fori_loop(0, _CAPC // _L, pad, 0, unroll=False)
        handles = [
            pltpu.async_copy(cols.at[pl.ds(k * 128, 128), :],
                             staged.at[blist3.at[k]], sem_s)
            for k in range(_CHK)
        ]
        for h in handles:
            h.wait()
        return 0

    def process(win, t, cnt2):
        def scan_i(i, c):
            uvec = ul[pl.ds(i * _L, _L)]
            m = (lax.shift_right_logical(uvec, 7) == t) & (
                iota + (i * _L) < n_match)
            plsc.store_compressed(todo.at[pl.ds(0, _L)], iota + (i * _L), mask=m)
            mcount = plsc.all_reduce_population_count(m)[0]

            def ext(k, c2):
                c2 = lax.cond(c2 + _L > _CAPC, flush, lambda x: x, c2)
                pos = todo[pl.ds(k, _L)][0]
                u = ul[pl.ds(pos, _L)][0]
                b = bl[pl.ds(pos, _L)][0]
                lane = jnp.broadcast_to(u & 127, (_L,))
                for g in range(_D // _L):
                    vals = plsc.load_gather(win, [iota + (g * _L), lane])
                    cols[c2, pl.ds(g * _L, _L)] = vals
                plsc.store_scatter(
                    blist3, [jnp.broadcast_to(c2 // 128, (_L,)),
                             jnp.broadcast_to(lax.rem(c2, 128), (_L,)) + iota],
                    jnp.broadcast_to(b, (_L,)), mask=iota == 0)
                return c2 + 1

            return lax.fori_loop(0, mcount, ext, c, unroll=False)

        return lax.fori_loop(0, n_scan, scan_i, cnt2, unroll=False)

    fetch(start, win0, sem0).wait()
    c1 = fetch(start + 1, win1, sem1)
    del c1

    def pair_step(pi, cnt2):
        t0 = start + 2 * pi
        cnt2 = process(win0, t0, cnt2)

        @pl.when(pi + 1 < _NPAIR)
        def _():
            fetch(t0 + 2, win0, sem0)

        pltpu.make_async_copy(tab_t.at[:, pl.ds(0, 128)], win1, sem1).wait()
        cnt2 = process(win1, t0 + 1, cnt2)

        @pl.when(pi + 1 < _NPAIR)
        def _():
            fetch(t0 + 3, win1, sem1)
            pltpu.make_async_copy(
                tab_t.at[:, pl.ds(0, 128)], win0, sem0).wait()

        return cnt2

    cnt2 = lax.fori_loop(0, _NPAIR, pair_step, 0, unroll=False)
    flush(cnt2)


def _stage1_body(users_hbm, items_hbm, utt_hbm, itt_hbm, su_hbm, si_hbm,
                 idx_all, win0, win1, ul, bl, todo, cols, blist3,
                 sem0, sem1, sem_s):
    wid = lax.axis_index("s") * _NC + lax.axis_index("c")
    start = wid * _BLKW
    _sweep(users_hbm, utt_hbm, su_hbm, idx_all, win0, win1, ul, bl, todo,
           cols, blist3, sem0, sem1, sem_s, start)
    _sweep(items_hbm, itt_hbm, si_hbm, idx_all, win0, win1, ul, bl, todo,
           cols, blist3, sem0, sem1, sem_s, start)


def _stage2_body(su_hbm, si_hbm, out_hbm, ub, ib, prod, sem_u, sem_i):
    wid = lax.axis_index("s") * _NC + lax.axis_index("c")
    base = wid * (_B // _NW)

    def chunk(ch, carry):
        row = base + ch * 128
        cu = pltpu.async_copy(su_hbm.at[pl.ds(row, 128), :], ub, sem_u)
        ci = pltpu.async_copy(si_hbm.at[pl.ds(row, 128), :], ib, sem_i)
        cu.wait()
        ci.wait()

        def mul_row(r, c2):
            for g in range(_D // _L):
                sl = pl.ds(g * _L, _L)
                prod[r, sl] = ub[r, sl] * ib[r, sl]
            return c2

        lax.fori_loop(0, 128, mul_row, 0, unroll=False)
        pltpu.sync_copy(prod, out_hbm.at[pl.ds(row, 128), :])
        return carry

    lax.fori_loop(0, _B // _NW // 128, chunk, 0, unroll=False)


@jax.jit
def _gmf(users, items, user_table_t, item_table_t):
    mesh = plsc.VectorSubcoreMesh(core_axis_name="c", subcore_axis_name="s")
    stage1 = pl.kernel(
        _stage1_body,
        out_type=(
            jax.ShapeDtypeStruct((_SPAD, 128), jnp.float32),
            jax.ShapeDtypeStruct((_SPAD, 128), jnp.float32),
        ),
        mesh=mesh,
        scratch_types=[
            pltpu.VMEM((_B,), jnp.int32),
            pltpu.VMEM((_D, 128), jnp.float32),
            pltpu.VMEM((_D, 128), jnp.float32),
            pltpu.VMEM((_B + _L,), jnp.int32),
            pltpu.VMEM((_B + _L,), jnp.int32),
            pltpu.VMEM((2 * _L,), jnp.int32),
            pltpu.VMEM((_CAPC, 128), jnp.float32),
            pltpu.VMEM((_CHK, 128), jnp.int32),
            pltpu.SemaphoreType.DMA,
            pltpu.SemaphoreType.DMA,
            pltpu.SemaphoreType.DMA,
        ],
        compiler_params=pltpu.CompilerParams(disable_bounds_checks=True, needs_layout_passes=False),
    )
    su, si = stage1(users, items, user_table_t, item_table_t)

    stage2 = pl.kernel(
        _stage2_body,
        out_type=jax.ShapeDtypeStruct((_B, _D), jnp.float32),
        mesh=mesh,
        scratch_types=[
            pltpu.VMEM((128, 128), jnp.float32),
            pltpu.VMEM((128, 128), jnp.float32),
            pltpu.VMEM((128, _D), jnp.float32),
            pltpu.SemaphoreType.DMA,
            pltpu.SemaphoreType.DMA,
        ],
        compiler_params=pltpu.CompilerParams(disable_bounds_checks=True, needs_layout_passes=False),
    )
    return stage2(su, si)


def kernel(users, items, user_table, item_table):
    return _gmf(users.astype(jnp.int32), items.astype(jnp.int32),
                user_table.T, item_table.T)


# speedup vs baseline: 1.3880x; 1.3880x over previous
import functools

import jax
import jax.numpy as jnp
from jax import lax
from jax.experimental import pallas as pl
from jax.experimental.pallas import tpu as pltpu
from jax.experimental.pallas import tpu_sc as plsc

_B = 16384
_D = 64
_L = 16
_NROW = 1000000
_NBLK = (_NROW + 127) // 128
_SPAD = _B + 8

_info = plsc.get_sparse_core_info()
_NC, _NS = _info.num_cores, _info.num_subcores
_NW = _NC * _NS
_BLKW = 245
_NPAIR = 123
_NB = 2 * _NPAIR
_CAPC = 256
_CHK = _CAPC // 128


def _one_lane(ref, idx, val, iota):
    plsc.store_scatter(ref, [jnp.broadcast_to(idx, (_L,))],
                       jnp.broadcast_to(val, (_L,)), mask=iota == 0)


def _sweep(idx_hbm, tab_t, staged, idx_all, win0, win1, su, sb, counts,
           begin, offs, todo, cols, blist3, sem0, sem1, sem_s, start):
    iota = lax.iota(jnp.int32, _L)

    pltpu.sync_copy(idx_hbm, idx_all.at[pl.ds(0, _B)])

    def zero(i, c):
        counts[pl.ds(i * _L, _L)] = jnp.broadcast_to(0, (_L,))
        return c

    lax.fori_loop(0, 256 // _L, zero, 0, unroll=False)

    def hist_step(j, cnt):
        uvec = idx_all[pl.ds(j * _L, _L)]
        blk = lax.shift_right_logical(uvec, 7)
        m = (blk >= start) & (blk < start + _NB)
        plsc.store_compressed(todo.at[pl.ds(0, _L)], blk - start, mask=m)
        mcount = plsc.all_reduce_population_count(m)[0]

        def inc(k, c2):
            bl = todo[pl.ds(k, _L)][0]
            c = counts[pl.ds(bl, _L)][0]
            _one_lane(counts, bl, c + 1, iota)
            return c2

        lax.fori_loop(0, mcount, inc, 0, unroll=False)
        return cnt + mcount

    n_match = lax.fori_loop(0, _B // _L, hist_step, 0, unroll=False)

    def prefix(i, carry):
        v = counts[pl.ds(i * _L, _L)]
        incl = plsc.cumsum(v)
        beg = incl - v + carry
        begin[pl.ds(i * _L, _L)] = beg
        offs[pl.ds(i * _L, _L)] = beg
        return carry + incl[_L - 1]

    lax.fori_loop(0, 256 // _L, prefix, 0, unroll=False)

    def place_step(j, c):
        uvec = idx_all[pl.ds(j * _L, _L)]
        blk = lax.shift_right_logical(uvec, 7)
        m = (blk >= start) & (blk < start + _NB)
        plsc.store_compressed(todo.at[pl.ds(0, _L)], iota + (j * _L), mask=m)
        mcount = plsc.all_reduce_population_count(m)[0]

        def put(k, c2):
            b = todo[pl.ds(k, _L)][0]
            u = idx_all[pl.ds(b, _L)][0]
            bl = lax.shift_right_logical(u, 7) - start
            pos = offs[pl.ds(bl, _L)][0]
            _one_lane(su, pos, u, iota)
            _one_lane(sb, pos, b, iota)
            _one_lane(offs, bl, pos + 1, iota)
            return c2

        lax.fori_loop(0, mcount, put, 0, unroll=False)
        return c

    lax.fori_loop(0, _B // _L, place_step, 0, unroll=False)

    def fetch(t, buf, sem):
        tc = jnp.minimum(t, _NBLK - 1)
        off = pl.multiple_of(tc * 128, 128)
        return pltpu.async_copy(tab_t.at[:, pl.ds(off, 128)], buf, sem)

    def flush(cnt2):
        def pad(i, c):
            k = i // (128 // _L)
            col = (i % (128 // _L)) * _L
            jvec = iota + (k * 128 + col)
            old = blist3[k, pl.ds(col, _L)]
            blist3[k, pl.ds(col, _L)] = jnp.where(jvec < cnt2, old, _B)
            return c

        lax.fori_loop(0, _CAPC // _L, pad, 0, unroll=False)
        handles = [
            pltpu.async_copy(cols.at[pl.ds(k * 128, 128), :],
                             staged.at[blist3.at[k]], sem_s)
            for k in range(_CHK)
        ]
        for h in handles:
            h.wait()
        return 0

    def process(win, t, cnt2):
        tl = t - start
        lo = begin[pl.ds(tl, _L)][0]
        hi = begin[pl.ds(tl + 1, _L)][0]

        def ext(k, c2):
            c2 = lax.cond(c2 >= _CAPC, flush, lambda x: x, c2)
            u = su[pl.ds(k, _L)][0]
            b = sb[pl.ds(k, _L)][0]
            lane = jnp.broadcast_to(u & 127, (_L,))
            for g in range(_D // _L):
                vals = plsc.load_gather(win, [iota + (g * _L), lane])
                cols[c2, pl.ds(g * _L, _L)] = vals
            plsc.store_scatter(
                blist3, [jnp.broadcast_to(c2 // 128, (_L,)),
                         jnp.broadcast_to(lax.rem(c2, 128), (_L,))],
                jnp.broadcast_to(b, (_L,)), mask=iota == 0)
            return c2 + 1

        return lax.fori_loop(lo, hi, ext, cnt2, unroll=False)

    fetch(start, win0, sem0).wait()
    fetch(start + 1, win1, sem1)

    def pair_step(pi, cnt2):
        t0 = start + 2 * pi
        cnt2 = process(win0, t0, cnt2)

        @pl.when(pi + 1 < _NPAIR)
        def _():
            fetch(t0 + 2, win0, sem0)

        pltpu.make_async_copy(tab_t.at[:, pl.ds(0, 128)], win1, sem1).wait()
        cnt2 = process(win1, t0 + 1, cnt2)

        @pl.when(pi + 1 < _NPAIR)
        def _():
            fetch(t0 + 3, win1, sem1)
            pltpu.make_async_copy(
                tab_t.at[:, pl.ds(0, 128)], win0, sem0).wait()

        return cnt2

    cnt2 = lax.fori_loop(0, _NPAIR, pair_step, 0, unroll=False)
    flush(cnt2)


def _stage1_body(users_hbm, items_hbm, utt_hbm, itt_hbm, su_hbm, si_hbm,
                 idx_all, win0, win1, su, sb, counts, begin, offs, todo,
                 cols, blist3, sem0, sem1, sem_s):
    wid = lax.axis_index("s") * _NC + lax.axis_index("c")
    start = wid * _BLKW
    _sweep(users_hbm, utt_hbm, su_hbm, idx_all, win0, win1, su, sb, counts,
           begin, offs, todo, cols, blist3, sem0, sem1, sem_s, start)
    _sweep(items_hbm, itt_hbm, si_hbm, idx_all, win0, win1, su, sb, counts,
           begin, offs, todo, cols, blist3, sem0, sem1, sem_s, start)


def _stage2_body(su_hbm, si_hbm, out_hbm, ub, ib, prod, sem_u, sem_i):
    wid = lax.axis_index("s") * _NC + lax.axis_index("c")
    base = wid * (_B // _NW)

    def chunk(ch, carry):
        row = base + ch * 128
        cu = pltpu.async_copy(su_hbm.at[pl.ds(row, 128), :], ub, sem_u)
        ci = pltpu.async_copy(si_hbm.at[pl.ds(row, 128), :], ib, sem_i)
        cu.wait()
        ci.wait()

        def mul_row(r, c2):
            for g in range(_D // _L):
                sl = pl.ds(g * _L, _L)
                prod[r, sl] = ub[r, sl] * ib[r, sl]
            return c2

        lax.fori_loop(0, 128, mul_row, 0, unroll=False)
        pltpu.sync_copy(prod, out_hbm.at[pl.ds(row, 128), :])
        return carry

    lax.fori_loop(0, _B // _NW // 128, chunk, 0, unroll=False)


@jax.jit
def _gmf(users, items, user_table_t, item_table_t):
    mesh = plsc.VectorSubcoreMesh(core_axis_name="c", subcore_axis_name="s")
    stage1 = pl.kernel(
        _stage1_body,
        out_type=(
            jax.ShapeDtypeStruct((_SPAD, 128), jnp.float32),
            jax.ShapeDtypeStruct((_SPAD, 128), jnp.float32),
        ),
        mesh=mesh,
        scratch_types=[
            pltpu.VMEM((_B + _L,), jnp.int32),
            pltpu.VMEM((_D, 128), jnp.float32),
            pltpu.VMEM((_D, 128), jnp.float32),
            pltpu.VMEM((_B + _L,), jnp.int32),
            pltpu.VMEM((_B + _L,), jnp.int32),
            pltpu.VMEM((256 + _L,), jnp.int32),
            pltpu.VMEM((256 + _L,), jnp.int32),
            pltpu.VMEM((256 + _L,), jnp.int32),
            pltpu.VMEM((2 * _L,), jnp.int32),
            pltpu.VMEM((_CAPC, 128), jnp.float32),
            pltpu.VMEM((_CHK, 128), jnp.int32),
            pltpu.SemaphoreType.DMA,
            pltpu.SemaphoreType.DMA,
            pltpu.SemaphoreType.DMA,
        ],
        compiler_params=pltpu.CompilerParams(
            disable_bounds_checks=True, needs_layout_passes=False),
    )
    su, si = stage1(users, items, user_table_t, item_table_t)

    stage2 = pl.kernel(
        _stage2_body,
        out_type=jax.ShapeDtypeStruct((_B, _D), jnp.float32),
        mesh=mesh,
        scratch_types=[
            pltpu.VMEM((128, 128), jnp.float32),
            pltpu.VMEM((128, 128), jnp.float32),
            pltpu.VMEM((128, _D), jnp.float32),
            pltpu.SemaphoreType.DMA,
            pltpu.SemaphoreType.DMA,
        ],
        compiler_params=pltpu.CompilerParams(
            disable_bounds_checks=True, needs_layout_passes=False),
    )
    return stage2(su, si)


def kernel(users, items, user_table, item_table):
    return _gmf(users.astype(jnp.int32), items.astype(jnp.int32),
                user_table.T, item_table.T)


# speedup vs baseline: 1.4856x; 1.0703x over previous
import functools

import jax
import jax.numpy as jnp
from jax import lax
from jax.experimental import pallas as pl
from jax.experimental.pallas import tpu as pltpu
from jax.experimental.pallas import tpu_sc as plsc

_B = 16384
_D = 64
_L = 16
_NROW = 1000000
_NBLK = (_NROW + 127) // 128
_SPAD = _B + 8

_info = plsc.get_sparse_core_info()
_NC, _NS = _info.num_cores, _info.num_subcores
_NW = _NC * _NS
_BLKW = 245
_NB = 248
_NQ = _NB // 4
_CAPC = 256
_CHK = _CAPC // 128


def _one_lane(ref, idx, val, iota):
    plsc.store_scatter(ref, [jnp.broadcast_to(idx, (_L,))],
                       jnp.broadcast_to(val, (_L,)), mask=iota == 0)


def _sweep(idx_hbm, tab_t, staged, idx_all, wins, su, sb, counts,
           begin, offs, todo, cols, blist3, sems, sem_s, start):
    iota = lax.iota(jnp.int32, _L)

    pltpu.sync_copy(idx_hbm, idx_all.at[pl.ds(0, _B)])

    def zero(i, c):
        counts[pl.ds(i * _L, _L)] = jnp.broadcast_to(0, (_L,))
        return c

    lax.fori_loop(0, 256 // _L, zero, 0, unroll=False)

    def hist_step(j, cnt):
        uvec = idx_all[pl.ds(j * _L, _L)]
        blk = lax.shift_right_logical(uvec, 7)
        m = (blk >= start) & (blk < start + _NB)
        plsc.store_compressed(todo.at[pl.ds(0, _L)], blk - start, mask=m)
        mcount = plsc.all_reduce_population_count(m)[0]

        def inc(k, c2):
            bl = todo[pl.ds(k, _L)][0]
            c = counts[pl.ds(bl, _L)][0]
            _one_lane(counts, bl, c + 1, iota)
            return c2

        lax.fori_loop(0, mcount, inc, 0, unroll=False)
        return cnt + mcount

    n_match = lax.fori_loop(0, _B // _L, hist_step, 0, unroll=False)

    def prefix(i, carry):
        v = counts[pl.ds(i * _L, _L)]
        incl = plsc.cumsum(v)
        beg = incl - v + carry
        begin[pl.ds(i * _L, _L)] = beg
        offs[pl.ds(i * _L, _L)] = beg
        return carry + incl[_L - 1]

    lax.fori_loop(0, 256 // _L, prefix, 0, unroll=False)

    def place_step(j, c):
        uvec = idx_all[pl.ds(j * _L, _L)]
        blk = lax.shift_right_logical(uvec, 7)
        m = (blk >= start) & (blk < start + _NB)
        plsc.store_compressed(todo.at[pl.ds(0, _L)], iota + (j * _L), mask=m)
        mcount = plsc.all_reduce_population_count(m)[0]

        def put(k, c2):
            b = todo[pl.ds(k, _L)][0]
            u = idx_all[pl.ds(b, _L)][0]
            bl = lax.shift_right_logical(u, 7) - start
            pos = offs[pl.ds(bl, _L)][0]
            _one_lane(su, pos, u, iota)
            _one_lane(sb, pos, b, iota)
            _one_lane(offs, bl, pos + 1, iota)
            return c2

        lax.fori_loop(0, mcount, put, 0, unroll=False)
        return c

    lax.fori_loop(0, _B // _L, place_step, 0, unroll=False)

    def fetch(t, buf, sem):
        tc = jnp.minimum(t, _NBLK - 1)
        off = pl.multiple_of(tc * 128, 128)
        return pltpu.async_copy(tab_t.at[:, pl.ds(off, 128)], buf, sem)

    def flush(cnt2):
        def pad(i, c):
            k = i // (128 // _L)
            col = (i % (128 // _L)) * _L
            jvec = iota + (k * 128 + col)
            old = blist3[k, pl.ds(col, _L)]
            blist3[k, pl.ds(col, _L)] = jnp.where(jvec < cnt2, old, _B)
            return c

        lax.fori_loop(0, _CAPC // _L, pad, 0, unroll=False)
        handles = [
            pltpu.async_copy(cols.at[pl.ds(k * 128, 128), :],
                             staged.at[blist3.at[k]], sem_s)
            for k in range(_CHK)
        ]
        for h in handles:
            h.wait()
        return 0

    def process(win, t, cnt2):
        tl = t - start
        lo = begin[pl.ds(tl, _L)][0]
        hi = begin[pl.ds(tl + 1, _L)][0]

        def ext(k, c2):
            c2 = lax.cond(c2 >= _CAPC, flush, lambda x: x, c2)
            u = su[pl.ds(k, _L)][0]
            b = sb[pl.ds(k, _L)][0]
            lane = jnp.broadcast_to(u & 127, (_L,))
            for g in range(_D // _L):
                vals = plsc.load_gather(win, [iota + (g * _L), lane])
                cols[c2, pl.ds(g * _L, _L)] = vals
            plsc.store_scatter(
                blist3, [jnp.broadcast_to(c2 // 128, (_L,)),
                         jnp.broadcast_to(lax.rem(c2, 128), (_L,))],
                jnp.broadcast_to(b, (_L,)), mask=iota == 0)
            return c2 + 1

        return lax.fori_loop(lo, hi, ext, cnt2, unroll=False)

    for j in range(4):
        fetch(start + j, wins[j], sems[j])

    def quad_step(qi, cnt2):
        t0 = start + qi * 4
        for j in range(4):
            pltpu.make_async_copy(
                tab_t.at[:, pl.ds(0, 128)], wins[j], sems[j]).wait()
            cnt2 = process(wins[j], t0 + j, cnt2)

            @pl.when(t0 + j + 4 < start + _NB)
            def _():
                fetch(t0 + j + 4, wins[j], sems[j])

        return cnt2

    cnt2 = lax.fori_loop(0, _NQ, quad_step, 0, unroll=False)
    flush(cnt2)


def _stage1_body(users_hbm, items_hbm, utt_hbm, itt_hbm, su_hbm, si_hbm,
                 idx_all, w0, w1, w2, w3, su, sb, counts, begin, offs, todo,
                 cols, blist3, s0, s1, s2, s3, sem_s):
    wid = lax.axis_index("s") * _NC + lax.axis_index("c")
    start = wid * _BLKW
    wins = [w0, w1, w2, w3]
    sems = [s0, s1, s2, s3]
    _sweep(users_hbm, utt_hbm, su_hbm, idx_all, wins, su, sb, counts,
           begin, offs, todo, cols, blist3, sems, sem_s, start)
    _sweep(items_hbm, itt_hbm, si_hbm, idx_all, wins, su, sb, counts,
           begin, offs, todo, cols, blist3, sems, sem_s, start)


def _stage2_body(su_hbm, si_hbm, out_hbm, ub, ib, prod, sem_u, sem_i):
    wid = lax.axis_index("s") * _NC + lax.axis_index("c")
    base = wid * (_B // _NW)

    def chunk(ch, carry):
        row = base + ch * 128
        cu = pltpu.async_copy(su_hbm.at[pl.ds(row, 128), :], ub, sem_u)
        ci = pltpu.async_copy(si_hbm.at[pl.ds(row, 128), :], ib, sem_i)
        cu.wait()
        ci.wait()

        def mul_row(r, c2):
            for g in range(_D // _L):
                sl = pl.ds(g * _L, _L)
                prod[r, sl] = ub[r, sl] * ib[r, sl]
            return c2

        lax.fori_loop(0, 128, mul_row, 0, unroll=False)
        pltpu.sync_copy(prod, out_hbm.at[pl.ds(row, 128), :])
        return carry

    lax.fori_loop(0, _B // _NW // 128, chunk, 0, unroll=False)


@jax.jit
def _gmf(users, items, user_table_t, item_table_t):
    mesh = plsc.VectorSubcoreMesh(core_axis_name="c", subcore_axis_name="s")
    stage1 = pl.kernel(
        _stage1_body,
        out_type=(
            jax.ShapeDtypeStruct((_SPAD, 128), jnp.float32),
            jax.ShapeDtypeStruct((_SPAD, 128), jnp.float32),
        ),
        mesh=mesh,
        scratch_types=[
            pltpu.VMEM((_B + _L,), jnp.int32),
            pltpu.VMEM((_D, 128), jnp.float32),
            pltpu.VMEM((_D, 128), jnp.float32),
            pltpu.VMEM((_D, 128), jnp.float32),
            pltpu.VMEM((_D, 128), jnp.float32),
            pltpu.VMEM((_B + _L,), jnp.int32),
            pltpu.VMEM((_B + _L,), jnp.int32),
            pltpu.VMEM((256 + _L,), jnp.int32),
            pltpu.VMEM((256 + _L,), jnp.int32),
            pltpu.VMEM((256 + _L,), jnp.int32),
            pltpu.VMEM((2 * _L,), jnp.int32),
            pltpu.VMEM((_CAPC, 128), jnp.float32),
            pltpu.VMEM((_CHK, 128), jnp.int32),
            pltpu.SemaphoreType.DMA,
            pltpu.SemaphoreType.DMA,
            pltpu.SemaphoreType.DMA,
            pltpu.SemaphoreType.DMA,
            pltpu.SemaphoreType.DMA,
        ],
        compiler_params=pltpu.CompilerParams(
            disable_bounds_checks=True, needs_layout_passes=False),
    )
    su, si = stage1(users, items, user_table_t, item_table_t)

    stage2 = pl.kernel(
        _stage2_body,
        out_type=jax.ShapeDtypeStruct((_B, _D), jnp.float32),
        mesh=mesh,
        scratch_types=[
            pltpu.VMEM((128, 128), jnp.float32),
            pltpu.VMEM((128, 128), jnp.float32),
            pltpu.VMEM((128, _D), jnp.float32),
            pltpu.SemaphoreType.DMA,
            pltpu.SemaphoreType.DMA,
        ],
        compiler_params=pltpu.CompilerParams(
            disable_bounds_checks=True, needs_layout_passes=False),
    )
    return stage2(su, si)


def kernel(users, items, user_table, item_table):
    return _gmf(users.astype(jnp.int32), items.astype(jnp.int32),
                user_table.T, item_table.T)


# speedup vs baseline: 1.6124x; 1.0854x over previous
import functools

import jax
import jax.numpy as jnp
from jax import lax
from jax.experimental import pallas as pl
from jax.experimental.pallas import tpu as pltpu
from jax.experimental.pallas import tpu_sc as plsc

_B = 16384
_D = 64
_L = 16
_NROW = 1000000
_NBLK = (_NROW + 127) // 128
_SPAD = _B + 8

_info = plsc.get_sparse_core_info()
_NC, _NS = _info.num_cores, _info.num_subcores
_NW = _NC * _NS
_BLKW = 245
_NB = 248
_NQ = _NB // 4
_CAPC = 256
_CHK = _CAPC // 128


def _one_lane(ref, idx, val, iota):
    plsc.store_scatter(ref, [jnp.broadcast_to(idx, (_L,))],
                       jnp.broadcast_to(val, (_L,)), mask=iota == 0)


def _sweep(idx_hbm, tab_t, staged, idx_all, wins, su, sb, counts16,
           off16, begin, cols, blist3, sems, sem_s, start):
    iota = lax.iota(jnp.int32, _L)
    ones = jnp.broadcast_to(1, (_L,))

    pltpu.sync_copy(idx_hbm, idx_all.at[pl.ds(0, _B)])

    def zero(i, c):
        counts16[pl.ds(i * _L, _L)] = jnp.broadcast_to(0, (_L,))
        return c

    lax.fori_loop(0, 256, zero, 0, unroll=False)

    def hist_step(j, c):
        uvec = idx_all[pl.ds(j * _L, _L)]
        blk = lax.shift_right_logical(uvec, 7)
        m = (blk >= start) & (blk < start + _NB)
        blkl = jnp.where(m, blk - start, 0)
        plsc.addupdate_scatter(counts16, [blkl * _L + iota], ones, mask=m)
        return c

    lax.fori_loop(0, _B // _L, hist_step, 0, unroll=False)

    def prefix(b, carry):
        v = counts16[pl.ds(b * _L, _L)]
        incl = plsc.cumsum(v)
        off16[pl.ds(b * _L, _L)] = incl - v + carry
        _one_lane(begin, b, carry, iota)
        return carry + incl[_L - 1]

    lax.fori_loop(0, 256, prefix, 0, unroll=False)

    def place_step(j, c):
        uvec = idx_all[pl.ds(j * _L, _L)]
        blk = lax.shift_right_logical(uvec, 7)
        m = (blk >= start) & (blk < start + _NB)
        blkl = jnp.where(m, blk - start, 0)
        addr = blkl * _L + iota
        pos = plsc.load_gather(off16, [addr])
        plsc.store_scatter(su, [pos], uvec, mask=m)
        plsc.store_scatter(sb, [pos], iota + (j * _L), mask=m)
        plsc.addupdate_scatter(off16, [addr], ones, mask=m)
        return c

    lax.fori_loop(0, _B // _L, place_step, 0, unroll=False)

    def fetch(t, buf, sem):
        tc = jnp.minimum(t, _NBLK - 1)
        off = pl.multiple_of(tc * 128, 128)
        return pltpu.async_copy(tab_t.at[:, pl.ds(off, 128)], buf, sem)

    def flush(cnt2):
        def pad(i, c):
            k = i // (128 // _L)
            col = (i % (128 // _L)) * _L
            jvec = iota + (k * 128 + col)
            old = blist3[k, pl.ds(col, _L)]
            blist3[k, pl.ds(col, _L)] = jnp.where(jvec < cnt2, old, _B)
            return c

        lax.fori_loop(0, _CAPC // _L, pad, 0, unroll=False)
        handles = [
            pltpu.async_copy(cols.at[pl.ds(k * 128, 128), :],
                             staged.at[blist3.at[k]], sem_s)
            for k in range(_CHK)
        ]
        for h in handles:
            h.wait()
        return 0

    def process(win, t, cnt2):
        tl = t - start
        lo = begin[pl.ds(tl, _L)][0]
        hi = begin[pl.ds(tl + 1, _L)][0]

        def ext(k, c2):
            c2 = lax.cond(c2 >= _CAPC, flush, lambda x: x, c2)
            u = su[pl.ds(k, _L)][0]
            b = sb[pl.ds(k, _L)][0]
            lane = jnp.broadcast_to(u & 127, (_L,))
            for g in range(_D // _L):
                vals = plsc.load_gather(win, [iota + (g * _L), lane])
                cols[c2, pl.ds(g * _L, _L)] = vals
            plsc.store_scatter(
                blist3, [jnp.broadcast_to(c2 // 128, (_L,)),
                         jnp.broadcast_to(lax.rem(c2, 128), (_L,))],
                jnp.broadcast_to(b, (_L,)), mask=iota == 0)
            return c2 + 1

        return lax.fori_loop(lo, hi, ext, cnt2, unroll=False)

    for j in range(4):
        fetch(start + j, wins[j], sems[j])

    def quad_step(qi, cnt2):
        t0 = start + qi * 4
        for j in range(4):
            pltpu.make_async_copy(
                tab_t.at[:, pl.ds(0, 128)], wins[j], sems[j]).wait()
            cnt2 = process(wins[j], t0 + j, cnt2)

            @pl.when(t0 + j + 4 < start + _NB)
            def _():
                fetch(t0 + j + 4, wins[j], sems[j])

        return cnt2

    cnt2 = lax.fori_loop(0, _NQ, quad_step, 0, unroll=False)
    flush(cnt2)


def _stage1_body(users_hbm, items_hbm, utt_hbm, itt_hbm, su_hbm, si_hbm,
                 idx_all, w0, w1, w2, w3, su, sb, counts16, off16, begin,
                 cols, blist3, s0, s1, s2, s3, sem_s):
    wid = lax.axis_index("s") * _NC + lax.axis_index("c")
    start = wid * _BLKW
    wins = [w0, w1, w2, w3]
    sems = [s0, s1, s2, s3]
    _sweep(users_hbm, utt_hbm, su_hbm, idx_all, wins, su, sb, counts16,
           off16, begin, cols, blist3, sems, sem_s, start)
    _sweep(items_hbm, itt_hbm, si_hbm, idx_all, wins, su, sb, counts16,
           off16, begin, cols, blist3, sems, sem_s, start)


def _stage2_body(su_hbm, si_hbm, out_hbm, ub, ib, prod, sem_u, sem_i):
    wid = lax.axis_index("s") * _NC + lax.axis_index("c")
    base = wid * (_B // _NW)

    def chunk(ch, carry):
        row = base + ch * 128
        cu = pltpu.async_copy(su_hbm.at[pl.ds(row, 128), :], ub, sem_u)
        ci = pltpu.async_copy(si_hbm.at[pl.ds(row, 128), :], ib, sem_i)
        cu.wait()
        ci.wait()

        def mul_row(r, c2):
            for g in range(_D // _L):
                sl = pl.ds(g * _L, _L)
                prod[r, sl] = ub[r, sl] * ib[r, sl]
            return c2

        lax.fori_loop(0, 128, mul_row, 0, unroll=False)
        pltpu.sync_copy(prod, out_hbm.at[pl.ds(row, 128), :])
        return carry

    lax.fori_loop(0, _B // _NW // 128, chunk, 0, unroll=False)


@jax.jit
def _gmf(users, items, user_table_t, item_table_t):
    mesh = plsc.VectorSubcoreMesh(core_axis_name="c", subcore_axis_name="s")
    stage1 = pl.kernel(
        _stage1_body,
        out_type=(
            jax.ShapeDtypeStruct((_SPAD, 128), jnp.float32),
            jax.ShapeDtypeStruct((_SPAD, 128), jnp.float32),
        ),
        mesh=mesh,
        scratch_types=[
            pltpu.VMEM((_B + _L,), jnp.int32),
            pltpu.VMEM((_D, 128), jnp.float32),
            pltpu.VMEM((_D, 128), jnp.float32),
            pltpu.VMEM((_D, 128), jnp.float32),
            pltpu.VMEM((_D, 128), jnp.float32),
            pltpu.VMEM((_B + _L,), jnp.int32),
            pltpu.VMEM((_B + _L,), jnp.int32),
            pltpu.VMEM((256 * _L,), jnp.int32),
            pltpu.VMEM((256 * _L,), jnp.int32),
            pltpu.VMEM((256 + _L,), jnp.int32),
            pltpu.VMEM((_CAPC, 128), jnp.float32),
            pltpu.VMEM((_CHK, 128), jnp.int32),
            pltpu.SemaphoreType.DMA,
            pltpu.SemaphoreType.DMA,
            pltpu.SemaphoreType.DMA,
            pltpu.SemaphoreType.DMA,
            pltpu.SemaphoreType.DMA,
        ],
        compiler_params=pltpu.CompilerParams(
            disable_bounds_checks=True, needs_layout_passes=False),
    )
    su, si = stage1(users, items, user_table_t, item_table_t)

    stage2 = pl.kernel(
        _stage2_body,
        out_type=jax.ShapeDtypeStruct((_B, _D), jnp.float32),
        mesh=mesh,
        scratch_types=[
            pltpu.VMEM((128, 128), jnp.float32),
            pltpu.VMEM((128, 128), jnp.float32),
            pltpu.VMEM((128, _D), jnp.float32),
            pltpu.SemaphoreType.DMA,
            pltpu.SemaphoreType.DMA,
        ],
        compiler_params=pltpu.CompilerParams(
            disable_bounds_checks=True, needs_layout_passes=False),
    )
    return stage2(su, si)


def kernel(users, items, user_table, item_table):
    return _gmf(users.astype(jnp.int32), items.astype(jnp.int32),
                user_table.T, item_table.T)
